# baseline (speedup 1.0000x reference)
import functools

import jax
import jax.numpy as jnp
from jax import lax
from jax.experimental import pallas as pl
from jax.experimental.pallas import tpu as pltpu
from jax.experimental.pallas import tpu_sc as plsc

B = 2
NW = 32
C = 96
G2 = B * 96 * 96
G1 = B * 192 * 96
T2_ROWS = B * 98 * 98
T1_ROWS = B * 196 * 98
PW2 = G2 // C // NW
PW1 = G1 // C // NW


def _build_rows2(idx2d, idxp):
    boff2 = (jnp.arange(B, dtype=jnp.int32) * (98 * 98))[:, None, None]
    y96 = jnp.arange(96, dtype=jnp.int32)
    rows2 = []
    for i in range(3):
        for j in range(3):
            lh = y96[:, None] + (1 - i)
            lw = y96[None, :] + (1 - j)
            valid = (lh >= 0) & (lh < 96) & (lw >= 0) & (lw < 96)
            p = idxp[:, 4 - i:100 - i, 4 - j:100 - j]
            ph, pw = p // 96, p % 96
            r = (ph + i) * 98 + (pw + j) + boff2
            rows2.append(jnp.where(valid[None], r, 0))
    rows2 = jnp.stack(rows2, axis=1)
    rows2 = rows2.transpose(0, 2, 3, 1).reshape(G2 // C, C, 9)
    return rows2.transpose(0, 2, 1)


def _build_rows1(idx2d, idxp):
    boff1 = (jnp.arange(B, dtype=jnp.int32) * (196 * 98))[:, None, None]
    y192 = jnp.arange(192, dtype=jnp.int32)
    m96 = jnp.arange(96, dtype=jnp.int32)
    rows1 = []
    for ki in range(3):
        for kj in range(3):
            lh = y192[:, None] // 2 + (1 - ki)
            lw = m96[None, :] + (1 - kj)
            valid = (lh >= 0) & (lh < 96) & (lw >= 0) & (lw < 96)
            s = idxp[:, 4 - ki:100 - ki, 4 - kj:100 - kj]
            p = jnp.repeat(s, 2, axis=1)
            ph, pw = p // 96, p % 96
            i = (y192 % 2)[:, None] + 2 * ki
            r = (2 * ph + i) * 98 + (pw + kj) + boff1
            rows1.append(jnp.where(valid[None], r, 0))
    rows1 = jnp.stack(rows1, axis=1)
    rows1 = rows1.transpose(0, 2, 3, 1).reshape(G1 // C, C, 9)
    return rows1.transpose(0, 2, 1)


def _make_sc_fold(g_rows, per_w):
    @functools.partial(
        pl.kernel,
        mesh=plsc.VectorSubcoreMesh(core_axis_name="c", subcore_axis_name="s"),
        out_type=jax.ShapeDtypeStruct((g_rows, 128), jnp.float32),
        scratch_types=[
            pltpu.VMEM((per_w, 9, C), jnp.int32),
            pltpu.VMEM((3, C, 128), jnp.float32),
            pltpu.SemaphoreType.DMA,
            pltpu.SemaphoreType.DMA,
            pltpu.SemaphoreType.DMA,
            pltpu.SemaphoreType.DMA,
            pltpu.SemaphoreType.DMA,
            pltpu.SemaphoreType.DMA,
            pltpu.SemaphoreType.DMA,
        ],
    )
    def fold(t_hbm, i_hbm, out_hbm, ixv, st, isem, g0, g1, g2, o0, o1, o2):
        wid = lax.axis_index("s") * 2 + lax.axis_index("c")
        ninth = jnp.float32(1.0 / 9.0)
        zero16 = jnp.zeros((16,), jnp.float32)
        gsems = (g0, g1, g2)
        osems = (o0, o1, o2)

        pltpu.async_copy(i_hbm.at[pl.ds(wid * per_w, per_w)], ixv, isem).wait()

        addcps = [None] * per_w
        outcps = [None] * per_w

        def start(t):
            s = t % 3
            if t >= 3:
                outcps[t - 3].wait()

            def zbody(r, rc):
                for c in range(8):
                    st[s, r, pl.ds(c * 16, 16)] = zero16
                return rc

            lax.fori_loop(0, C, zbody, 0)
            addcps[t] = [
                pltpu.async_copy(
                    t_hbm.at[ixv.at[t].at[k]], st.at[s], gsems[s], add=True)
                for k in range(9)
            ]

        def finish(t):
            s = t % 3
            for cp in addcps[t]:
                cp.wait()

            def rowbody(r, rc):
                for c in range(8):
                    v = st[s, r, pl.ds(c * 16, 16)]
                    st[s, r, pl.ds(c * 16, 16)] = v * ninth
                return rc

            lax.fori_loop(0, C, rowbody, 0)
            outcps[t] = pltpu.async_copy(
                st.at[s], out_hbm.at[pl.ds((wid * per_w + t) * C, C)],
                osems[s])

        start(0)
        start(1)
        for t in range(per_w):
            if t + 2 < per_w:
                start(t + 2)
            finish(t)
        outcps[per_w - 3].wait()
        outcps[per_w - 2].wait()
        outcps[per_w - 1].wait()

    return fold


_fold2 = _make_sc_fold(G2, PW2)
_fold1 = _make_sc_fold(G1, PW1)


def kernel(R_lv2_star_arg, lrsr_lv2, ref_lv1, ref_lv2):
    del lrsr_lv2
    idx2d = R_lv2_star_arg.astype(jnp.int32).reshape(B, 96, 96)
    idxp = jnp.pad(idx2d, ((0, 0), (3, 3), (3, 3)))

    rows2 = _build_rows2(idx2d, idxp)
    t2 = jnp.pad(ref_lv2, ((0, 0), (0, 0), (1, 1), (1, 1)))
    t2 = t2.transpose(0, 2, 3, 1).reshape(T2_ROWS, 128)
    out2_rows = _fold2(t2, rows2)

    rows1 = _build_rows1(idx2d, idxp)
    t1 = jnp.pad(ref_lv1, ((0, 0), (0, 0), (2, 2), (2, 2)))
    t1 = t1.transpose(0, 2, 3, 1).reshape(T1_ROWS, 128)
    out1_rows = _fold1(t1, rows1)

    T_lv2 = out2_rows.reshape(B, 96, 96, 128).transpose(0, 3, 1, 2)
    T_lv1 = out1_rows.reshape(B, 192, 96, 2, 64).reshape(
        B, 192, 192, 64).transpose(0, 3, 1, 2)
    return (T_lv2, T_lv1)


# speedup vs baseline: 1.0722x; 1.0722x over previous
import functools

import jax
import jax.numpy as jnp
from jax import lax
from jax.experimental import pallas as pl
from jax.experimental.pallas import tpu as pltpu
from jax.experimental.pallas import tpu_sc as plsc

B = 2
NW = 32
C = 96
G2 = B * 96 * 96
G1 = B * 192 * 96
T2_ROWS = B * 98 * 98
T1_ROWS = B * 196 * 98
PW2 = G2 // C // NW
PW1 = G1 // C // NW


def _build_rows(idx):
    idx2d = idx.astype(jnp.int32).reshape(B, 96, 96)
    idxp = jnp.pad(idx2d, ((0, 0), (3, 3), (3, 3)))
    boff2 = (jnp.arange(B, dtype=jnp.int32) * (98 * 98))[:, None, None]
    boff1 = (jnp.arange(B, dtype=jnp.int32) * (196 * 98))[:, None, None]

    y96 = jnp.arange(96, dtype=jnp.int32)
    rows2 = []
    for i in range(3):
        for j in range(3):
            lh = y96[:, None] + (1 - i)
            lw = y96[None, :] + (1 - j)
            valid = (lh >= 0) & (lh < 96) & (lw >= 0) & (lw < 96)
            p = idxp[:, 4 - i:100 - i, 4 - j:100 - j]
            ph, pw = p // 96, p % 96
            r = (ph + i) * 98 + (pw + j) + boff2
            rows2.append(jnp.where(valid[None], r, 0))
    rows2 = jnp.stack(rows2, axis=1)
    rows2 = rows2.transpose(0, 2, 3, 1).reshape(G2 // C, C, 9)
    rows2 = rows2.transpose(0, 2, 1)

    y192 = jnp.arange(192, dtype=jnp.int32)
    m96 = jnp.arange(96, dtype=jnp.int32)
    rows1 = []
    for ki in range(3):
        for kj in range(3):
            lh = y192[:, None] // 2 + (1 - ki)
            lw = m96[None, :] + (1 - kj)
            valid = (lh >= 0) & (lh < 96) & (lw >= 0) & (lw < 96)
            s = idxp[:, 4 - ki:100 - ki, 4 - kj:100 - kj]
            p = jnp.repeat(s, 2, axis=1)
            ph, pw = p // 96, p % 96
            i = (y192 % 2)[:, None] + 2 * ki
            r = (2 * ph + i) * 98 + (pw + kj) + boff1
            rows1.append(jnp.where(valid[None], r, 0))
    rows1 = jnp.stack(rows1, axis=1)
    rows1 = rows1.transpose(0, 2, 3, 1).reshape(G1 // C, C, 9)
    rows1 = rows1.transpose(0, 2, 1)
    return rows2, rows1


@functools.partial(
    pl.kernel,
    mesh=plsc.VectorSubcoreMesh(core_axis_name="c", subcore_axis_name="s"),
    out_type=(
        jax.ShapeDtypeStruct((G2, 128), jnp.float32),
        jax.ShapeDtypeStruct((G1, 128), jnp.float32),
    ),
    scratch_types=[
        pltpu.VMEM((PW2, 9, C), jnp.int32),
        pltpu.VMEM((PW1, 9, C), jnp.int32),
        pltpu.VMEM((3, C, 128), jnp.float32),
        pltpu.SemaphoreType.DMA,
        pltpu.SemaphoreType.DMA,
        pltpu.SemaphoreType.DMA,
        pltpu.SemaphoreType.DMA,
        pltpu.SemaphoreType.DMA,
        pltpu.SemaphoreType.DMA,
        pltpu.SemaphoreType.DMA,
    ],
)
def _sc_transfer(t2_hbm, i2_hbm, t1_hbm, i1_hbm, out2_hbm, out1_hbm,
                 ix2, ix1, st, isem, g0, g1, g2, o0, o1, o2):
    wid = lax.axis_index("s") * 2 + lax.axis_index("c")
    ninth = jnp.float32(1.0 / 9.0)
    zero16 = jnp.zeros((16,), jnp.float32)
    gsems = (g0, g1, g2)
    osems = (o0, o1, o2)

    pltpu.async_copy(i2_hbm.at[pl.ds(wid * PW2, PW2)], ix2, isem).wait()
    pltpu.async_copy(i1_hbm.at[pl.ds(wid * PW1, PW1)], ix1, isem).wait()

    def phase(t_hbm, ixv, out_hbm, per_w):
        addcps = [None] * per_w
        outcps = [None] * per_w

        def start(t):
            s = t % 3
            if t >= 3:
                outcps[t - 3].wait()

            def zbody(r, rc):
                for c in range(8):
                    st[s, r, pl.ds(c * 16, 16)] = zero16
                return rc

            lax.fori_loop(0, C, zbody, 0)
            addcps[t] = [
                pltpu.async_copy(
                    t_hbm.at[ixv.at[t].at[k]], st.at[s], gsems[s], add=True)
                for k in range(9)
            ]

        def finish(t):
            s = t % 3
            for cp in addcps[t]:
                cp.wait()

            def rowbody(r, rc):
                for c in range(8):
                    v = st[s, r, pl.ds(c * 16, 16)]
                    st[s, r, pl.ds(c * 16, 16)] = v * ninth
                return rc

            lax.fori_loop(0, C, rowbody, 0)
            outcps[t] = pltpu.async_copy(
                st.at[s], out_hbm.at[pl.ds((wid * per_w + t) * C, C)],
                osems[s])

        start(0)
        start(1)
        for t in range(per_w):
            if t + 2 < per_w:
                start(t + 2)
            finish(t)
        outcps[per_w - 3].wait()
        outcps[per_w - 2].wait()
        outcps[per_w - 1].wait()

    phase(t2_hbm, ix2, out2_hbm, PW2)
    phase(t1_hbm, ix1, out1_hbm, PW1)


def kernel(R_lv2_star_arg, lrsr_lv2, ref_lv1, ref_lv2):
    del lrsr_lv2
    rows2, rows1 = _build_rows(R_lv2_star_arg)
    t2 = jnp.pad(ref_lv2, ((0, 0), (0, 0), (1, 1), (1, 1)))
    t2 = t2.transpose(0, 2, 3, 1).reshape(T2_ROWS, 128)
    t1 = jnp.pad(ref_lv1, ((0, 0), (0, 0), (2, 2), (2, 2)))
    t1 = t1.transpose(0, 2, 3, 1).reshape(T1_ROWS, 128)

    out2_rows, out1_rows = _sc_transfer(t2, rows2, t1, rows1)

    T_lv2 = out2_rows.reshape(B, 96, 96, 128).transpose(0, 3, 1, 2)
    T_lv1 = out1_rows.reshape(B, 192, 96, 2, 64).reshape(
        B, 192, 192, 64).transpose(0, 3, 1, 2)
    return (T_lv2, T_lv1)
